# Initial kernel scaffold; baseline (speedup 1.0000x reference)
import jax
import jax.numpy as jnp
from jax.experimental import pallas as pl


def kernel(x, sampling_rate, f0, pad_to):
    raise NotImplementedError("write your pallas kernel here")


# speedup vs baseline: 426.7890x; 426.7890x over previous
import functools

import jax
import jax.numpy as jnp
from jax import lax
from jax.experimental import pallas as pl
from jax.experimental.pallas import tpu as pltpu
from jax.experimental.pallas import tpu_sc as plsc

NC = 2
NS = 16
L = 16
SRC = 524288
PAD_N = 1048576
P = 8192
NPER = PAD_N // P
BIG = 1 << 29

K1_BP = P // NS
K1_G = K1_BP // L
K2_OUT = PAD_N // (NC * NS)
K2_PER = K2_OUT // P

_mesh = plsc.VectorSubcoreMesh(
    core_axis_name="c", subcore_axis_name="s", num_cores=NC, num_subcores=NS)


def _k1_body(f0_hbm, blkf_hbm, blki_hbm, scalf_hbm, scali_hbm,
             f0c_v, v_v, kp_v, kn_v, sums_loc, v8k_v, sums_v,
             a_v, b_v, pv_v, nv_v, scalf_v, scali_v,
             v8k_sh, sums_sh):
    cid = lax.axis_index("c")
    sid = lax.axis_index("s")
    w = sid
    iota = lax.iota(jnp.int32, L)

    @pl.when(cid == 0)
    def _local():
        f0_base = jnp.where(w < 8, w * 256, 520192 + w * 256)
        pltpu.sync_copy(f0_hbm.at[pl.ds(f0_base, 256)], f0c_v)

        def expand(g, _):
            lm = g * L + iota
            v_v[pl.ds(g * L, L)] = plsc.load_gather(f0c_v, [lm >> 1])
            return 0
        lax.fori_loop(0, K1_G, expand, 0)

        def fwd(g, carry):
            prevk, firstk, cntv = carry
            v = v_v[pl.ds(g * L, L)]
            m = v != 0.0
            kg = iota + (g * L + w * K1_BP)
            pm = jnp.maximum(plsc.cummax(jnp.where(m, kg, -1)), prevk)
            kp_v[pl.ds(g * L, L)] = pm
            firstk = jnp.minimum(firstk, jnp.min(jnp.where(m, kg, BIG)))
            return jnp.max(pm), firstk, cntv + m.astype(jnp.int32)

        lastk, firstk, cntv = lax.fori_loop(
            0, K1_G, fwd,
            (jnp.int32(-1), jnp.int32(BIG), jnp.zeros((L,), jnp.int32)))
        cnt = jnp.sum(cntv)

        def bwd(t, nextk):
            g = K1_G - 1 - t
            v = v_v[pl.ds(g * L, L)]
            m = v != 0.0
            kg = iota + (g * L + w * K1_BP)
            nin = jnp.where(m, kg, BIG)
            suf = -lax.rev(plsc.cummax(lax.rev(-nin, (0,))), (0,))
            nk = jnp.minimum(suf, nextk)
            kn_v[pl.ds(g * L, L)] = nk
            return jnp.min(nk)
        lax.fori_loop(0, K1_G, bwd, jnp.int32(BIG))

        sums_loc[pl.ds(0, L)] = jnp.broadcast_to(lastk, (L,))
        sums_loc[pl.ds(L, L)] = jnp.broadcast_to(firstk, (L,))
        sums_loc[pl.ds(2 * L, L)] = jnp.broadcast_to(cnt, (L,))
        pltpu.sync_copy(v_v, v8k_sh.at[pl.ds(w * K1_BP, K1_BP)])
        pltpu.sync_copy(sums_loc, sums_sh.at[pl.ds(w * 3 * L, 3 * L)])

    plsc.subcore_barrier()

    @pl.when(cid == 0)
    def _combine():
        pltpu.sync_copy(v8k_sh, v8k_v)
        pltpu.sync_copy(sums_sh, sums_v)

        def comb(j, carry):
            bpk, bnk, gfirst, glast, gcnt = carry
            lk = jnp.max(sums_v[pl.ds(j * 3 * L, L)])
            fk = jnp.min(sums_v[pl.ds(j * 3 * L + L, L)])
            ct = jnp.max(sums_v[pl.ds(j * 3 * L + 2 * L, L)])
            has = lk >= 0
            bpk = jnp.where(has & (j < w), lk, bpk)
            bnk = jnp.where(has & (j > w) & (bnk >= BIG), fk, bnk)
            return (bpk, bnk, jnp.minimum(gfirst, fk),
                    jnp.maximum(glast, lk), gcnt + ct)

        bpk, bnk, gfirst, glast, gcnt = lax.fori_loop(
            0, NS, comb,
            (jnp.int32(-1), jnp.int32(BIG), jnp.int32(BIG),
             jnp.int32(-1), jnp.int32(0)))

        def res(g, _):
            kg = iota + (g * L + w * K1_BP)
            kp = kp_v[pl.ds(g * L, L)]
            kp = jnp.where(kp >= 0, kp, bpk)
            kp = jnp.where(kp >= 0, kp, glast - P)
            kn = kn_v[pl.ds(g * L, L)]
            kn = jnp.where(kn < BIG, kn, bnk)
            kn = jnp.where(kn < BIG, kn, gfirst + P)
            a_v[pl.ds(g * L, L)] = kg - kp
            b_v[pl.ds(g * L, L)] = kn - kg
            pv_v[pl.ds(g * L, L)] = plsc.load_gather(v8k_v, [(kp + P) & (P - 1)])
            nv_v[pl.ds(g * L, L)] = plsc.load_gather(v8k_v, [kn & (P - 1)])
            return 0
        lax.fori_loop(0, K1_G, res, 0)

        pltpu.sync_copy(v_v, blkf_hbm.at[pl.ds(w * K1_BP, K1_BP)])
        pltpu.sync_copy(pv_v, blkf_hbm.at[pl.ds(P + w * K1_BP, K1_BP)])
        pltpu.sync_copy(nv_v, blkf_hbm.at[pl.ds(2 * P + w * K1_BP, K1_BP)])
        pltpu.sync_copy(a_v, blki_hbm.at[pl.ds(w * K1_BP, K1_BP)])
        pltpu.sync_copy(b_v, blki_hbm.at[pl.ds(P + w * K1_BP, K1_BP)])

        @pl.when(w == 0)
        def _scal():
            lidx = jnp.broadcast_to(jnp.clip(gfirst, 0, P - 1), (L,))
            ridx = jnp.broadcast_to(jnp.clip(glast, 0, P - 1), (L,))
            scalf_v[pl.ds(0, L)] = plsc.load_gather(v8k_v, [lidx])
            scalf_v[pl.ds(L, L)] = plsc.load_gather(v8k_v, [ridx])
            scali_v[pl.ds(0, L)] = jnp.broadcast_to(gcnt, (L,))
            pltpu.sync_copy(scalf_v, scalf_hbm)
            pltpu.sync_copy(scali_v, scali_hbm)


def _k2_body(par_hbm, blkf_hbm, blki_hbm, scalf_hbm, scali_hbm, out_hbm,
             v_v, pv_v, nv_v, a_v, b_v, out_v, par_v, scalf_v, scali_v):
    cid = lax.axis_index("c")
    sid = lax.axis_index("s")
    wid = sid * NC + cid
    base = wid * K2_OUT
    iota = lax.iota(jnp.int32, L)

    pltpu.sync_copy(blkf_hbm.at[pl.ds(0, P)], v_v)
    pltpu.sync_copy(blkf_hbm.at[pl.ds(P, P)], pv_v)
    pltpu.sync_copy(blkf_hbm.at[pl.ds(2 * P, P)], nv_v)
    pltpu.sync_copy(blki_hbm.at[pl.ds(0, P)], a_v)
    pltpu.sync_copy(blki_hbm.at[pl.ds(P, P)], b_v)
    pltpu.sync_copy(par_hbm, par_v)
    pltpu.sync_copy(scalf_hbm, scalf_v)
    pltpu.sync_copy(scali_hbm, scali_v)

    c_vec = par_v[pl.ds(0, L)]
    sr_vec = par_v[pl.ds(L, L)]
    leftv = scalf_v[pl.ds(0, L)]
    rightv = scalf_v[pl.ds(L, L)]
    iszero = scali_v[pl.ds(0, L)] == 0

    for p in range(K2_PER):
        ibase = base + p * P

        def grp(g, _, ibase=ibase, off=p * P):
            s = g * L
            v = v_v[pl.ds(s, L)]
            a = a_v[pl.ds(s, L)]
            b = b_v[pl.ds(s, L)]
            pv = pv_v[pl.ds(s, L)]
            nv = nv_v[pl.ds(s, L)]
            ivec = iota + (ibase + s)
            pp = ivec - a
            np_ = ivec + b
            ti = (ivec.astype(jnp.float32) * 512.0) / sr_vec
            tp = c_vec * pp.astype(jnp.float32)
            tn = c_vec * np_.astype(jnp.float32)
            o = (pv * (tn - ti) + nv * (ti - tp)) / (tn - tp)
            m = v != 0.0
            o = jnp.where(m, v, o)
            nm = ~m
            o = jnp.where(nm & (pp < 0), leftv, o)
            o = jnp.where(nm & (np_ >= PAD_N), rightv, o)
            o = jnp.where(iszero, 0.0, o)
            out_v[pl.ds(off + s, L)] = o
            return 0

        lax.fori_loop(0, P // L, grp, 0)

    pltpu.sync_copy(out_v, out_hbm.at[pl.ds(base, K2_OUT)])


_k1 = functools.partial(
    pl.kernel, _k1_body,
    out_type=(jax.ShapeDtypeStruct((3 * P,), jnp.float32),
              jax.ShapeDtypeStruct((2 * P,), jnp.int32),
              jax.ShapeDtypeStruct((2 * L,), jnp.float32),
              jax.ShapeDtypeStruct((L,), jnp.int32)),
    mesh=_mesh,
    compiler_params=pltpu.CompilerParams(needs_layout_passes=False),
    scratch_types=[
        pltpu.VMEM((256,), jnp.float32),
        pltpu.VMEM((K1_BP,), jnp.float32),
        pltpu.VMEM((K1_BP,), jnp.int32),
        pltpu.VMEM((K1_BP,), jnp.int32),
        pltpu.VMEM((3 * L,), jnp.int32),
        pltpu.VMEM((P,), jnp.float32),
        pltpu.VMEM((NS * 3 * L,), jnp.int32),
        pltpu.VMEM((K1_BP,), jnp.int32),
        pltpu.VMEM((K1_BP,), jnp.int32),
        pltpu.VMEM((K1_BP,), jnp.float32),
        pltpu.VMEM((K1_BP,), jnp.float32),
        pltpu.VMEM((2 * L,), jnp.float32),
        pltpu.VMEM((L,), jnp.int32),
        pltpu.VMEM_SHARED((P,), jnp.float32),
        pltpu.VMEM_SHARED((NS * 3 * L,), jnp.int32),
    ])()

_k2 = functools.partial(
    pl.kernel, _k2_body,
    out_type=jax.ShapeDtypeStruct((PAD_N,), jnp.float32),
    mesh=_mesh,
    compiler_params=pltpu.CompilerParams(needs_layout_passes=False),
    scratch_types=[
        pltpu.VMEM((P,), jnp.float32),
        pltpu.VMEM((P,), jnp.float32),
        pltpu.VMEM((P,), jnp.float32),
        pltpu.VMEM((P,), jnp.int32),
        pltpu.VMEM((P,), jnp.int32),
        pltpu.VMEM((K2_OUT,), jnp.float32),
        pltpu.VMEM((2 * L,), jnp.float32),
        pltpu.VMEM((2 * L,), jnp.float32),
        pltpu.VMEM((L,), jnp.int32),
    ])()


def kernel(x, sampling_rate, f0, pad_to):
    del x, pad_to
    srf = jnp.asarray(sampling_rate).astype(jnp.float32)
    c = (512 / jnp.asarray(sampling_rate)).astype(jnp.float32)
    par = jnp.concatenate([jnp.full((L,), c, jnp.float32),
                           jnp.full((L,), srf, jnp.float32)])
    blkf, blki, scalf, scali = _k1(f0)
    return _k2(par, blkf, blki, scalf, scali)


# speedup vs baseline: 885.6052x; 2.0750x over previous
import functools

import jax
import jax.numpy as jnp
from jax import lax
from jax.experimental import pallas as pl
from jax.experimental.pallas import tpu as pltpu
from jax.experimental.pallas import tpu_sc as plsc

NC = 2
NS = 16
L = 16
SRC = 524288
PAD_N = 1048576
P = 8192
NPER = PAD_N // P
BIG = 1 << 29

K1_BP = P // NS
K1_G = K1_BP // L
K2_OUT = PAD_N // (NC * NS)
K2_PER = K2_OUT // P

_mesh = plsc.VectorSubcoreMesh(
    core_axis_name="c", subcore_axis_name="s", num_cores=NC, num_subcores=NS)


def _k1_body(f0_hbm, blkf_hbm, blki_hbm, scalf_hbm, scali_hbm,
             f0c_v, v_v, kp_v, kn_v, sums_loc, v8k_v, sums_v,
             a_v, b_v, pv_v, nv_v, scalf_v, scali_v,
             v8k_sh, sums_sh):
    cid = lax.axis_index("c")
    sid = lax.axis_index("s")
    w = sid
    iota = lax.iota(jnp.int32, L)

    @pl.when(cid == 0)
    def _local():
        f0_base = jnp.where(w < 8, w * 256, 520192 + w * 256)
        pltpu.sync_copy(f0_hbm.at[pl.ds(f0_base, 256)], f0c_v)

        def expand(g, _):
            lm = g * L + iota
            v_v[pl.ds(g * L, L)] = plsc.load_gather(f0c_v, [lm >> 1])
            return 0
        lax.fori_loop(0, K1_G, expand, 0)

        def fwd(g, carry):
            prevk, firstk, cntv = carry
            v = v_v[pl.ds(g * L, L)]
            m = v != 0.0
            kg = iota + (g * L + w * K1_BP)
            pm = jnp.maximum(plsc.cummax(jnp.where(m, kg, -1)), prevk)
            kp_v[pl.ds(g * L, L)] = pm
            firstk = jnp.minimum(firstk, jnp.min(jnp.where(m, kg, BIG)))
            return jnp.max(pm), firstk, cntv + m.astype(jnp.int32)

        lastk, firstk, cntv = lax.fori_loop(
            0, K1_G, fwd,
            (jnp.int32(-1), jnp.int32(BIG), jnp.zeros((L,), jnp.int32)))
        cnt = jnp.sum(cntv)

        def bwd(t, nextk):
            g = K1_G - 1 - t
            v = v_v[pl.ds(g * L, L)]
            m = v != 0.0
            kg = iota + (g * L + w * K1_BP)
            nin = jnp.where(m, kg, BIG)
            suf = -lax.rev(plsc.cummax(lax.rev(-nin, (0,))), (0,))
            nk = jnp.minimum(suf, nextk)
            kn_v[pl.ds(g * L, L)] = nk
            return jnp.min(nk)
        lax.fori_loop(0, K1_G, bwd, jnp.int32(BIG))

        sums_loc[pl.ds(0, L)] = jnp.broadcast_to(lastk, (L,))
        sums_loc[pl.ds(L, L)] = jnp.broadcast_to(firstk, (L,))
        sums_loc[pl.ds(2 * L, L)] = jnp.broadcast_to(cnt, (L,))
        pltpu.sync_copy(v_v, v8k_sh.at[pl.ds(w * K1_BP, K1_BP)])
        pltpu.sync_copy(sums_loc, sums_sh.at[pl.ds(w * 3 * L, 3 * L)])

    plsc.subcore_barrier()

    @pl.when(cid == 0)
    def _combine():
        pltpu.sync_copy(v8k_sh, v8k_v)
        pltpu.sync_copy(sums_sh, sums_v)

        def comb(j, carry):
            bpk, bnk, gfirst, glast, gcnt = carry
            lk = jnp.max(sums_v[pl.ds(j * 3 * L, L)])
            fk = jnp.min(sums_v[pl.ds(j * 3 * L + L, L)])
            ct = jnp.max(sums_v[pl.ds(j * 3 * L + 2 * L, L)])
            has = lk >= 0
            bpk = jnp.where(has & (j < w), lk, bpk)
            bnk = jnp.where(has & (j > w) & (bnk >= BIG), fk, bnk)
            return (bpk, bnk, jnp.minimum(gfirst, fk),
                    jnp.maximum(glast, lk), gcnt + ct)

        bpk, bnk, gfirst, glast, gcnt = lax.fori_loop(
            0, NS, comb,
            (jnp.int32(-1), jnp.int32(BIG), jnp.int32(BIG),
             jnp.int32(-1), jnp.int32(0)))

        pltpu.sync_copy(v_v, blkf_hbm.at[pl.ds(w * K1_BP, K1_BP)])

        @pl.when(w == 0)
        def _scal():
            lidx = jnp.broadcast_to(jnp.clip(gfirst, 0, P - 1), (L,))
            ridx = jnp.broadcast_to(jnp.clip(glast, 0, P - 1), (L,))
            scalf_v[pl.ds(0, L)] = plsc.load_gather(v8k_v, [lidx])
            scalf_v[pl.ds(L, L)] = plsc.load_gather(v8k_v, [ridx])
            scali_v[pl.ds(0, L)] = jnp.broadcast_to(gcnt, (L,))
            pltpu.sync_copy(scalf_v, scalf_hbm)
            pltpu.sync_copy(scali_v, scali_hbm)

        @pl.when(gcnt < P)
        def _zero_run_tables():
            _k1_resolve(w, iota, bpk, bnk, gfirst, glast,
                        kp_v, kn_v, v8k_v, a_v, b_v, pv_v, nv_v,
                        blkf_hbm, blki_hbm)


def _k1_resolve(w, iota, bpk, bnk, gfirst, glast,
                kp_v, kn_v, v8k_v, a_v, b_v, pv_v, nv_v,
                blkf_hbm, blki_hbm):
        def res(g, _):
            kg = iota + (g * L + w * K1_BP)
            kp = kp_v[pl.ds(g * L, L)]
            kp = jnp.where(kp >= 0, kp, bpk)
            kp = jnp.where(kp >= 0, kp, glast - P)
            kn = kn_v[pl.ds(g * L, L)]
            kn = jnp.where(kn < BIG, kn, bnk)
            kn = jnp.where(kn < BIG, kn, gfirst + P)
            a_v[pl.ds(g * L, L)] = kg - kp
            b_v[pl.ds(g * L, L)] = kn - kg
            pv_v[pl.ds(g * L, L)] = plsc.load_gather(v8k_v, [(kp + P) & (P - 1)])
            nv_v[pl.ds(g * L, L)] = plsc.load_gather(v8k_v, [kn & (P - 1)])
            return 0
        lax.fori_loop(0, K1_G, res, 0)

        pltpu.sync_copy(pv_v, blkf_hbm.at[pl.ds(P + w * K1_BP, K1_BP)])
        pltpu.sync_copy(nv_v, blkf_hbm.at[pl.ds(2 * P + w * K1_BP, K1_BP)])
        pltpu.sync_copy(a_v, blki_hbm.at[pl.ds(w * K1_BP, K1_BP)])
        pltpu.sync_copy(b_v, blki_hbm.at[pl.ds(P + w * K1_BP, K1_BP)])


def _k2_body(par_hbm, blkf_hbm, blki_hbm, scalf_hbm, scali_hbm, out_hbm,
             v_v, pv_v, nv_v, a_v, b_v, out_v, par_v, scalf_v, scali_v):
    cid = lax.axis_index("c")
    sid = lax.axis_index("s")
    wid = sid * NC + cid
    base = wid * K2_OUT
    iota = lax.iota(jnp.int32, L)

    pltpu.sync_copy(blkf_hbm.at[pl.ds(0, P)], v_v)
    pltpu.sync_copy(scali_hbm, scali_v)
    gcnt = jnp.max(scali_v[pl.ds(0, L)])

    @pl.when(gcnt == P)
    def _fast():
        for p in range(K2_PER):
            pltpu.sync_copy(v_v, out_hbm.at[pl.ds(base + p * P, P)])

    @pl.when(gcnt < P)
    def _slow():
        pltpu.sync_copy(blkf_hbm.at[pl.ds(P, P)], pv_v)
        pltpu.sync_copy(blkf_hbm.at[pl.ds(2 * P, P)], nv_v)
        pltpu.sync_copy(blki_hbm.at[pl.ds(0, P)], a_v)
        pltpu.sync_copy(blki_hbm.at[pl.ds(P, P)], b_v)
        pltpu.sync_copy(par_hbm, par_v)
        pltpu.sync_copy(scalf_hbm, scalf_v)

        c_vec = par_v[pl.ds(0, L)]
        sr_vec = par_v[pl.ds(L, L)]
        leftv = scalf_v[pl.ds(0, L)]
        rightv = scalf_v[pl.ds(L, L)]
        iszero = scali_v[pl.ds(0, L)] == 0

        for p in range(K2_PER):
            ibase = base + p * P

            def grp(g, _, ibase=ibase, off=p * P):
                s = g * L
                v = v_v[pl.ds(s, L)]
                a = a_v[pl.ds(s, L)]
                b = b_v[pl.ds(s, L)]
                pv = pv_v[pl.ds(s, L)]
                nv = nv_v[pl.ds(s, L)]
                ivec = iota + (ibase + s)
                pp = ivec - a
                np_ = ivec + b
                ti = (ivec.astype(jnp.float32) * 512.0) / sr_vec
                tp = c_vec * pp.astype(jnp.float32)
                tn = c_vec * np_.astype(jnp.float32)
                o = (pv * (tn - ti) + nv * (ti - tp)) / (tn - tp)
                m = v != 0.0
                o = jnp.where(m, v, o)
                nm = ~m
                o = jnp.where(nm & (pp < 0), leftv, o)
                o = jnp.where(nm & (np_ >= PAD_N), rightv, o)
                o = jnp.where(iszero, 0.0, o)
                out_v[pl.ds(off + s, L)] = o
                return 0

            lax.fori_loop(0, P // L, grp, 0)

        pltpu.sync_copy(out_v, out_hbm.at[pl.ds(base, K2_OUT)])


_k1 = functools.partial(
    pl.kernel, _k1_body,
    out_type=(jax.ShapeDtypeStruct((3 * P,), jnp.float32),
              jax.ShapeDtypeStruct((2 * P,), jnp.int32),
              jax.ShapeDtypeStruct((2 * L,), jnp.float32),
              jax.ShapeDtypeStruct((L,), jnp.int32)),
    mesh=_mesh,
    compiler_params=pltpu.CompilerParams(needs_layout_passes=False),
    scratch_types=[
        pltpu.VMEM((256,), jnp.float32),
        pltpu.VMEM((K1_BP,), jnp.float32),
        pltpu.VMEM((K1_BP,), jnp.int32),
        pltpu.VMEM((K1_BP,), jnp.int32),
        pltpu.VMEM((3 * L,), jnp.int32),
        pltpu.VMEM((P,), jnp.float32),
        pltpu.VMEM((NS * 3 * L,), jnp.int32),
        pltpu.VMEM((K1_BP,), jnp.int32),
        pltpu.VMEM((K1_BP,), jnp.int32),
        pltpu.VMEM((K1_BP,), jnp.float32),
        pltpu.VMEM((K1_BP,), jnp.float32),
        pltpu.VMEM((2 * L,), jnp.float32),
        pltpu.VMEM((L,), jnp.int32),
        pltpu.VMEM_SHARED((P,), jnp.float32),
        pltpu.VMEM_SHARED((NS * 3 * L,), jnp.int32),
    ])()

_k2 = functools.partial(
    pl.kernel, _k2_body,
    out_type=jax.ShapeDtypeStruct((PAD_N,), jnp.float32),
    mesh=_mesh,
    compiler_params=pltpu.CompilerParams(needs_layout_passes=False),
    scratch_types=[
        pltpu.VMEM((P,), jnp.float32),
        pltpu.VMEM((P,), jnp.float32),
        pltpu.VMEM((P,), jnp.float32),
        pltpu.VMEM((P,), jnp.int32),
        pltpu.VMEM((P,), jnp.int32),
        pltpu.VMEM((K2_OUT,), jnp.float32),
        pltpu.VMEM((2 * L,), jnp.float32),
        pltpu.VMEM((2 * L,), jnp.float32),
        pltpu.VMEM((L,), jnp.int32),
    ])()


def kernel(x, sampling_rate, f0, pad_to):
    del x, pad_to
    srf = jnp.asarray(sampling_rate).astype(jnp.float32)
    c = (512 / jnp.asarray(sampling_rate)).astype(jnp.float32)
    par = jnp.concatenate([jnp.full((L,), c, jnp.float32),
                           jnp.full((L,), srf, jnp.float32)])
    blkf, blki, scalf, scali = _k1(f0)
    return _k2(par, blkf, blki, scalf, scali)


# speedup vs baseline: 1135.7095x; 1.2824x over previous
import functools

import jax
import jax.numpy as jnp
from jax import lax
from jax.experimental import pallas as pl
from jax.experimental.pallas import tpu as pltpu
from jax.experimental.pallas import tpu_sc as plsc

NC = 2
NS = 16
L = 16
SRC = 524288
PAD_N = 1048576
P = 8192
BIG = 1 << 29

K1_BP = P // NS
K1_G = K1_BP // L
OUT_W = PAD_N // (NC * NS)
OUT_PER = OUT_W // P

_mesh = plsc.VectorSubcoreMesh(
    core_axis_name="c", subcore_axis_name="s", num_cores=NC, num_subcores=NS)


def _body(par_hbm, f0_hbm, out_hbm,
          f0c_v, v_v, kp_v, kn_v, sums_loc, v8k_v, sums_v,
          af_v, bf_v, pvf_v, nvf_v, out_v, par_v,
          v8k_sh, sums_sh, a_sh, b_sh, pv_sh, nv_sh):
    cid = lax.axis_index("c")
    sid = lax.axis_index("s")
    w = sid
    wid = sid * NC + cid
    obase = wid * OUT_W
    iota = lax.iota(jnp.int32, L)

    f0_base = jnp.where(w < 8, w * 256, 520192 + w * 256)
    pltpu.sync_copy(f0_hbm.at[pl.ds(f0_base, 256)], f0c_v)

    def expand(g, _):
        lm = g * L + iota
        v_v[pl.ds(g * L, L)] = plsc.load_gather(f0c_v, [lm >> 1])
        return 0
    lax.fori_loop(0, K1_G, expand, 0)

    def fwd(g, carry):
        prevk, firstk, cntv = carry
        v = v_v[pl.ds(g * L, L)]
        m = v != 0.0
        kg = iota + (g * L + w * K1_BP)
        pm = jnp.maximum(plsc.cummax(jnp.where(m, kg, -1)), prevk)
        kp_v[pl.ds(g * L, L)] = pm
        firstk = jnp.minimum(firstk, jnp.min(jnp.where(m, kg, BIG)))
        return jnp.max(pm), firstk, cntv + m.astype(jnp.int32)

    lastk, firstk, cntv = lax.fori_loop(
        0, K1_G, fwd,
        (jnp.int32(-1), jnp.int32(BIG), jnp.zeros((L,), jnp.int32)))
    cnt = jnp.sum(cntv)

    def bwd(t, nextk):
        g = K1_G - 1 - t
        v = v_v[pl.ds(g * L, L)]
        m = v != 0.0
        kg = iota + (g * L + w * K1_BP)
        nin = jnp.where(m, kg, BIG)
        suf = -lax.rev(plsc.cummax(lax.rev(-nin, (0,))), (0,))
        nk = jnp.minimum(suf, nextk)
        kn_v[pl.ds(g * L, L)] = nk
        return jnp.min(nk)
    lax.fori_loop(0, K1_G, bwd, jnp.int32(BIG))

    sums_loc[pl.ds(0, L)] = jnp.broadcast_to(lastk, (L,))
    sums_loc[pl.ds(L, L)] = jnp.broadcast_to(firstk, (L,))
    sums_loc[pl.ds(2 * L, L)] = jnp.broadcast_to(cnt, (L,))
    pltpu.sync_copy(v_v, v8k_sh.at[pl.ds(w * K1_BP, K1_BP)])
    pltpu.sync_copy(sums_loc, sums_sh.at[pl.ds(w * 3 * L, 3 * L)])

    plsc.subcore_barrier()

    pltpu.sync_copy(v8k_sh, v8k_v)
    pltpu.sync_copy(sums_sh, sums_v)

    def comb(j, carry):
        bpk, bnk, gfirst, glast, gcnt = carry
        lk = jnp.max(sums_v[pl.ds(j * 3 * L, L)])
        fk = jnp.min(sums_v[pl.ds(j * 3 * L + L, L)])
        ct = jnp.max(sums_v[pl.ds(j * 3 * L + 2 * L, L)])
        has = lk >= 0
        bpk = jnp.where(has & (j < w), lk, bpk)
        bnk = jnp.where(has & (j > w) & (bnk >= BIG), fk, bnk)
        return (bpk, bnk, jnp.minimum(gfirst, fk),
                jnp.maximum(glast, lk), gcnt + ct)

    bpk, bnk, gfirst, glast, gcnt = lax.fori_loop(
        0, NS, comb,
        (jnp.int32(-1), jnp.int32(BIG), jnp.int32(BIG),
         jnp.int32(-1), jnp.int32(0)))

    @pl.when(gcnt == P)
    def _fast():
        for p in range(OUT_PER):
            pltpu.sync_copy(v8k_v, out_hbm.at[pl.ds(obase + p * P, P)])

    @pl.when(gcnt < P)
    def _slow():
        def res(g, _):
            kg = iota + (g * L + w * K1_BP)
            kp = kp_v[pl.ds(g * L, L)]
            kp = jnp.where(kp >= 0, kp, bpk)
            kp = jnp.where(kp >= 0, kp, glast - P)
            kn = kn_v[pl.ds(g * L, L)]
            kn = jnp.where(kn < BIG, kn, bnk)
            kn = jnp.where(kn < BIG, kn, gfirst + P)
            af_v[pl.ds(w * K1_BP + g * L, L)] = kg - kp
            bf_v[pl.ds(w * K1_BP + g * L, L)] = kn - kg
            pvf_v[pl.ds(w * K1_BP + g * L, L)] = plsc.load_gather(
                v8k_v, [(kp + P) & (P - 1)])
            nvf_v[pl.ds(w * K1_BP + g * L, L)] = plsc.load_gather(
                v8k_v, [kn & (P - 1)])
            return 0
        lax.fori_loop(0, K1_G, res, 0)

        pltpu.sync_copy(af_v.at[pl.ds(w * K1_BP, K1_BP)],
                        a_sh.at[pl.ds(w * K1_BP, K1_BP)])
        pltpu.sync_copy(bf_v.at[pl.ds(w * K1_BP, K1_BP)],
                        b_sh.at[pl.ds(w * K1_BP, K1_BP)])
        pltpu.sync_copy(pvf_v.at[pl.ds(w * K1_BP, K1_BP)],
                        pv_sh.at[pl.ds(w * K1_BP, K1_BP)])
        pltpu.sync_copy(nvf_v.at[pl.ds(w * K1_BP, K1_BP)],
                        nv_sh.at[pl.ds(w * K1_BP, K1_BP)])
        plsc.subcore_barrier()
        pltpu.sync_copy(a_sh, af_v)
        pltpu.sync_copy(b_sh, bf_v)
        pltpu.sync_copy(pv_sh, pvf_v)
        pltpu.sync_copy(nv_sh, nvf_v)
        pltpu.sync_copy(par_hbm, par_v)

        c_vec = par_v[pl.ds(0, L)]
        sr_vec = par_v[pl.ds(L, L)]
        lidx = jnp.broadcast_to(jnp.clip(gfirst, 0, P - 1), (L,))
        ridx = jnp.broadcast_to(jnp.clip(glast, 0, P - 1), (L,))
        leftv = plsc.load_gather(v8k_v, [lidx])
        rightv = plsc.load_gather(v8k_v, [ridx])
        iszero = jnp.broadcast_to(gcnt, (L,)) == 0

        for p in range(OUT_PER):
            ibase = obase + p * P

            def grp(g, _, ibase=ibase, off=p * P):
                s = g * L
                v = v8k_v[pl.ds(s, L)]
                a = af_v[pl.ds(s, L)]
                b = bf_v[pl.ds(s, L)]
                pv = pvf_v[pl.ds(s, L)]
                nv = nvf_v[pl.ds(s, L)]
                ivec = iota + (ibase + s)
                pp = ivec - a
                np_ = ivec + b
                ti = (ivec.astype(jnp.float32) * 512.0) / sr_vec
                tp = c_vec * pp.astype(jnp.float32)
                tn = c_vec * np_.astype(jnp.float32)
                o = (pv * (tn - ti) + nv * (ti - tp)) / (tn - tp)
                m = v != 0.0
                o = jnp.where(m, v, o)
                nm = ~m
                o = jnp.where(nm & (pp < 0), leftv, o)
                o = jnp.where(nm & (np_ >= PAD_N), rightv, o)
                o = jnp.where(iszero, 0.0, o)
                out_v[pl.ds(off + s, L)] = o
                return 0

            lax.fori_loop(0, P // L, grp, 0)

        pltpu.sync_copy(out_v, out_hbm.at[pl.ds(obase, OUT_W)])


_kern = functools.partial(
    pl.kernel, _body,
    out_type=jax.ShapeDtypeStruct((PAD_N,), jnp.float32),
    mesh=_mesh,
    compiler_params=pltpu.CompilerParams(needs_layout_passes=False),
    scratch_types=[
        pltpu.VMEM((256,), jnp.float32),
        pltpu.VMEM((K1_BP,), jnp.float32),
        pltpu.VMEM((K1_BP,), jnp.int32),
        pltpu.VMEM((K1_BP,), jnp.int32),
        pltpu.VMEM((3 * L,), jnp.int32),
        pltpu.VMEM((P,), jnp.float32),
        pltpu.VMEM((NS * 3 * L,), jnp.int32),
        pltpu.VMEM((P,), jnp.int32),
        pltpu.VMEM((P,), jnp.int32),
        pltpu.VMEM((P,), jnp.float32),
        pltpu.VMEM((P,), jnp.float32),
        pltpu.VMEM((OUT_W,), jnp.float32),
        pltpu.VMEM((2 * L,), jnp.float32),
        pltpu.VMEM_SHARED((P,), jnp.float32),
        pltpu.VMEM_SHARED((NS * 3 * L,), jnp.int32),
        pltpu.VMEM_SHARED((P,), jnp.int32),
        pltpu.VMEM_SHARED((P,), jnp.int32),
        pltpu.VMEM_SHARED((P,), jnp.float32),
        pltpu.VMEM_SHARED((P,), jnp.float32),
    ])()


def kernel(x, sampling_rate, f0, pad_to):
    del x, pad_to
    srf = jnp.asarray(sampling_rate).astype(jnp.float32)
    c = (512 / jnp.asarray(sampling_rate)).astype(jnp.float32)
    par = jnp.concatenate([jnp.full((L,), c, jnp.float32),
                           jnp.full((L,), srf, jnp.float32)])
    return _kern(par, f0)
